# baseline (speedup 1.0000x reference)
import functools

import jax
import jax.numpy as jnp
from jax import lax
from jax.experimental import pallas as pl
from jax.experimental.pallas import tpu as pltpu
from jax.experimental.pallas import tpu_sc as plsc

_B, _S, _E = 64, 1024, 1024
_T = 768
_NE = 16
_K = 2

_NSC = 8
_TB = _B - _NSC
_NBUF = 4
_CB = 2
_NCH = _TB // _CB

_NW = 32
_WPB = _NW // _NSC
_RPW = _S // _WPB
_RCH = 32
_NCHS = _RPW // _RCH


def _sc_partial_sums_kernel(x_hbm, out_hbm, buf0, buf1, acc_v, sem0, sem1):
    wid = lax.axis_index("s") * 2 + lax.axis_index("c")
    b = _TB + wid // _WPB
    r0 = (wid % _WPB) * _RPW

    def zero(g, carry):
        acc_v[pl.ds(g * 16, 16)] = jnp.zeros((16,), jnp.float32)
        return carry

    lax.fori_loop(0, _E // 16, zero, 0)

    pltpu.make_async_copy(x_hbm.at[b, pl.ds(r0, _RCH)], buf0, sem0).start()
    pltpu.make_async_copy(x_hbm.at[b, pl.ds(r0 + _RCH, _RCH)], buf1, sem1).start()

    def accumulate(buf):
        def gbody(g, carry):
            base = g * 16
            v = acc_v[pl.ds(base, 16)]
            for r in range(_RCH):
                v = v + buf[r, pl.ds(base, 16)]
            acc_v[pl.ds(base, 16)] = v
            return carry

        lax.fori_loop(0, _E // 16, gbody, 0)

    def pair(k, carry):
        for half, (bf, sm) in enumerate(((buf0, sem0), (buf1, sem1))):
            c = 2 * k + half
            pltpu.make_async_copy(
                x_hbm.at[b, pl.ds(r0 + c * _RCH, _RCH)], bf, sm).wait()
            accumulate(bf)
            nc = c + 2

            @pl.when(nc < _NCHS)
            def _():
                pltpu.make_async_copy(
                    x_hbm.at[b, pl.ds(r0 + nc * _RCH, _RCH)], bf, sm).start()
        return carry

    lax.fori_loop(0, _NCHS // 2, pair, 0)
    pltpu.sync_copy(acc_v, out_hbm.at[wid % _WPB, wid // _WPB])


_sc_partial_sums = functools.partial(
    pl.kernel,
    mesh=plsc.VectorSubcoreMesh(core_axis_name="c", subcore_axis_name="s"),
    out_type=jax.ShapeDtypeStruct((_WPB, _NSC, _E), jnp.float32),
    scratch_types=[
        pltpu.VMEM((_RCH, _E), jnp.float32),
        pltpu.VMEM((_RCH, _E), jnp.float32),
        pltpu.VMEM((_E,), jnp.float32),
        pltpu.SemaphoreType.DMA,
        pltpu.SemaphoreType.DMA,
    ],
)(_sc_partial_sums_kernel)


def _gate_kernel(x_hbm, text_ref, w1_hbm, b1_ref, w2_ref, b2_ref, sc_ref,
                 w_out_ref, i_out_ref, l_out_ref,
                 buf, w1_v, acc_ref, sems, w1_sem):
    for r in range(_NBUF):
        pltpu.make_async_copy(
            x_hbm.at[pl.ds(r * _CB, _CB)], buf.at[r], sems.at[r]).start()
    pltpu.make_async_copy(w1_hbm, w1_v, w1_sem).start()

    def outer(o, carry):
        for r in range(_NBUF):
            c = o * _NBUF + r
            pltpu.make_async_copy(
                x_hbm.at[pl.ds(c * _CB, _CB)], buf.at[r], sems.at[r]).wait()
            s = jnp.sum(buf[r], axis=1)
            for q in range(_CB):
                acc_ref[pl.ds(c * _CB + q, 1), :] = s[q:q + 1]
            nc = c + _NBUF

            @pl.when(nc < _NCH)
            def _():
                pltpu.make_async_copy(
                    x_hbm.at[pl.ds(nc * _CB, _CB)], buf.at[r], sems.at[r]).start()
        return carry

    jax.lax.fori_loop(0, _NCH // _NBUF, outer, 0)

    p = sc_ref[...]
    s_sc = p[0]
    for q in range(1, _WPB):
        s_sc = s_sc + p[q]
    acc_ref[_TB:_B, :] = s_sc

    pltpu.make_async_copy(w1_hbm, w1_v, w1_sem).wait()
    mean = acc_ref[...] * (1.0 / _S)
    text = text_ref[...]
    w1a = w1_v[0:_E, :]
    w1b = w1_v[_E:_E + _T, :]
    h = jnp.dot(mean, w1a, preferred_element_type=jnp.float32)
    h = h + jnp.dot(text, w1b, preferred_element_type=jnp.float32)
    h = jnp.maximum(h + b1_ref[...], 0.0)
    logits = (jnp.dot(h, w2_ref[...], preferred_element_type=jnp.float32)
              + b2_ref[...])
    l_out_ref[...] = logits

    lane = jax.lax.broadcasted_iota(jnp.int32, (_B, _NE), 1)
    m1 = jnp.max(logits, axis=1, keepdims=True)
    i1 = jnp.min(jnp.where(logits == m1, lane, _NE), axis=1, keepdims=True)
    masked = jnp.where(lane == i1, -jnp.inf, logits)
    m2 = jnp.max(masked, axis=1, keepdims=True)
    i2 = jnp.min(jnp.where(masked == m2, lane, _NE), axis=1, keepdims=True)

    lane2 = jax.lax.broadcasted_iota(jnp.int32, (_B, _K), 1)
    i_out_ref[...] = jnp.where(lane2 == 0, i1, i2)
    e2 = jnp.exp(m2 - m1)
    denom = 1.0 + e2
    w_out_ref[...] = jnp.where(lane2 == 0, 1.0 / denom, e2 / denom)


def kernel(x, text_embedding, W1, b1, W2, b2):
    b1r = b1.reshape(1, _E)
    b2r = b2.reshape(1, _NE)
    sc_partials = _sc_partial_sums(x)
    out_shape = (
        jax.ShapeDtypeStruct((_B, _K), jnp.float32),
        jax.ShapeDtypeStruct((_B, _K), jnp.int32),
        jax.ShapeDtypeStruct((_B, _NE), jnp.float32),
    )
    weights, indices, logits = pl.pallas_call(
        _gate_kernel,
        in_specs=[
            pl.BlockSpec(memory_space=pl.ANY),
            pl.BlockSpec(memory_space=pltpu.MemorySpace.VMEM),
            pl.BlockSpec(memory_space=pl.ANY),
            pl.BlockSpec(memory_space=pltpu.MemorySpace.VMEM),
            pl.BlockSpec(memory_space=pltpu.MemorySpace.VMEM),
            pl.BlockSpec(memory_space=pltpu.MemorySpace.VMEM),
            pl.BlockSpec(memory_space=pltpu.MemorySpace.VMEM),
        ],
        out_specs=(
            pl.BlockSpec(memory_space=pltpu.MemorySpace.VMEM),
            pl.BlockSpec(memory_space=pltpu.MemorySpace.VMEM),
            pl.BlockSpec(memory_space=pltpu.MemorySpace.VMEM),
        ),
        out_shape=out_shape,
        scratch_shapes=[
            pltpu.VMEM((_NBUF, _CB, _S, _E), jnp.float32),
            pltpu.VMEM((_E + _T, _E), jnp.float32),
            pltpu.VMEM((_B, _E), jnp.float32),
            pltpu.SemaphoreType.DMA((_NBUF,)),
            pltpu.SemaphoreType.DMA,
        ],
    )(x, text_embedding, W1, b1r, W2, b2r, sc_partials)
    return (weights, indices, logits)


# speedup vs baseline: 1.1054x; 1.1054x over previous
import functools

import jax
import jax.numpy as jnp
from jax import lax
from jax.experimental import pallas as pl
from jax.experimental.pallas import tpu as pltpu
from jax.experimental.pallas import tpu_sc as plsc

_B, _S, _E = 64, 1024, 1024
_T = 768
_NE = 16
_K = 2

_NSC = 16
_TB = _B - _NSC
_NBUF = 4
_CB = 2
_NCH = _TB // _CB

_NW = 32
_WPB = _NW // _NSC
_RPW = _S // _WPB
_RCH = 32
_NCHS = _RPW // _RCH


def _sc_partial_sums_kernel(x_hbm, out_hbm, buf0, buf1, acc_v, sem0, sem1):
    wid = lax.axis_index("s") * 2 + lax.axis_index("c")
    b = _TB + wid // _WPB
    r0 = (wid % _WPB) * _RPW

    def zero(g, carry):
        acc_v[pl.ds(g * 16, 16)] = jnp.zeros((16,), jnp.float32)
        return carry

    lax.fori_loop(0, _E // 16, zero, 0)

    pltpu.make_async_copy(x_hbm.at[b, pl.ds(r0, _RCH)], buf0, sem0).start()
    pltpu.make_async_copy(x_hbm.at[b, pl.ds(r0 + _RCH, _RCH)], buf1, sem1).start()

    def accumulate(buf):
        def gbody(g, carry):
            base = g * 16
            v = acc_v[pl.ds(base, 16)]
            for r in range(_RCH):
                v = v + buf[r, pl.ds(base, 16)]
            acc_v[pl.ds(base, 16)] = v
            return carry

        lax.fori_loop(0, _E // 16, gbody, 0)

    def pair(k, carry):
        for half, (bf, sm) in enumerate(((buf0, sem0), (buf1, sem1))):
            c = 2 * k + half
            pltpu.make_async_copy(
                x_hbm.at[b, pl.ds(r0 + c * _RCH, _RCH)], bf, sm).wait()
            accumulate(bf)
            nc = c + 2

            @pl.when(nc < _NCHS)
            def _():
                pltpu.make_async_copy(
                    x_hbm.at[b, pl.ds(r0 + nc * _RCH, _RCH)], bf, sm).start()
        return carry

    lax.fori_loop(0, _NCHS // 2, pair, 0)
    pltpu.sync_copy(acc_v, out_hbm.at[wid % _WPB, wid // _WPB])


_sc_partial_sums = functools.partial(
    pl.kernel,
    mesh=plsc.VectorSubcoreMesh(core_axis_name="c", subcore_axis_name="s"),
    out_type=jax.ShapeDtypeStruct((_WPB, _NSC, _E), jnp.float32),
    scratch_types=[
        pltpu.VMEM((_RCH, _E), jnp.float32),
        pltpu.VMEM((_RCH, _E), jnp.float32),
        pltpu.VMEM((_E,), jnp.float32),
        pltpu.SemaphoreType.DMA,
        pltpu.SemaphoreType.DMA,
    ],
)(_sc_partial_sums_kernel)


def _stream_kernel(x_hbm, text_ref, w1_hbm, b1_ref,
                   h_tc_ref, ht_sc_ref,
                   buf, w1_v, acc_ref, sems, w1_sem):
    for r in range(_NBUF):
        pltpu.make_async_copy(
            x_hbm.at[pl.ds(r * _CB, _CB)], buf.at[r], sems.at[r]).start()
    pltpu.make_async_copy(w1_hbm, w1_v, w1_sem).start()

    def outer(o, carry):
        for r in range(_NBUF):
            c = o * _NBUF + r
            pltpu.make_async_copy(
                x_hbm.at[pl.ds(c * _CB, _CB)], buf.at[r], sems.at[r]).wait()
            s = jnp.sum(buf[r], axis=1)
            for q in range(_CB):
                acc_ref[pl.ds(c * _CB + q, 1), :] = s[q:q + 1]
            nc = c + _NBUF

            @pl.when(nc < _NCH)
            def _():
                pltpu.make_async_copy(
                    x_hbm.at[pl.ds(nc * _CB, _CB)], buf.at[r], sems.at[r]).start()
        return carry

    jax.lax.fori_loop(0, _NCH // _NBUF, outer, 0)

    pltpu.make_async_copy(w1_hbm, w1_v, w1_sem).wait()
    mean = acc_ref[...] * (1.0 / _S)
    w1a = w1_v[0:_E, :]
    w1b = w1_v[_E:_E + _T, :]
    ht = (jnp.dot(text_ref[...], w1b, preferred_element_type=jnp.float32)
          + b1_ref[...])
    h = jnp.dot(mean, w1a, preferred_element_type=jnp.float32) + ht[0:_TB]
    h_tc_ref[...] = jnp.maximum(h, 0.0)
    ht_sc_ref[...] = ht[_TB:_B]


def _epilogue_kernel(h_tc_ref, ht_sc_ref, p_ref, w1_hbm, w2_ref, b2_ref,
                     w_out_ref, i_out_ref, l_out_ref, w1a_v, w1_sem):
    pltpu.make_async_copy(w1_hbm.at[pl.ds(0, _E)], w1a_v, w1_sem).start()
    s_sc = p_ref[0]
    for q in range(1, _WPB):
        s_sc = s_sc + p_ref[q]
    mean_sc = s_sc * (1.0 / _S)
    pltpu.make_async_copy(w1_hbm.at[pl.ds(0, _E)], w1a_v, w1_sem).wait()
    h_sc = (jnp.dot(mean_sc, w1a_v[...], preferred_element_type=jnp.float32)
            + ht_sc_ref[...])
    h_sc = jnp.maximum(h_sc, 0.0)

    logits_tc = jnp.dot(h_tc_ref[...], w2_ref[...],
                        preferred_element_type=jnp.float32)
    logits_sc = jnp.dot(h_sc, w2_ref[...],
                        preferred_element_type=jnp.float32)
    logits = jnp.concatenate([logits_tc, logits_sc], axis=0) + b2_ref[...]
    l_out_ref[...] = logits

    lane = jax.lax.broadcasted_iota(jnp.int32, (_B, _NE), 1)
    m1 = jnp.max(logits, axis=1, keepdims=True)
    i1 = jnp.min(jnp.where(logits == m1, lane, _NE), axis=1, keepdims=True)
    masked = jnp.where(lane == i1, -jnp.inf, logits)
    m2 = jnp.max(masked, axis=1, keepdims=True)
    i2 = jnp.min(jnp.where(masked == m2, lane, _NE), axis=1, keepdims=True)

    lane2 = jax.lax.broadcasted_iota(jnp.int32, (_B, _K), 1)
    i_out_ref[...] = jnp.where(lane2 == 0, i1, i2)
    e2 = jnp.exp(m2 - m1)
    denom = 1.0 + e2
    w_out_ref[...] = jnp.where(lane2 == 0, 1.0 / denom, e2 / denom)


def kernel(x, text_embedding, W1, b1, W2, b2):
    b1r = b1.reshape(1, _E)
    b2r = b2.reshape(1, _NE)

    sc_partials = _sc_partial_sums(x)

    h_tc, ht_sc = pl.pallas_call(
        _stream_kernel,
        in_specs=[
            pl.BlockSpec(memory_space=pl.ANY),
            pl.BlockSpec(memory_space=pltpu.MemorySpace.VMEM),
            pl.BlockSpec(memory_space=pl.ANY),
            pl.BlockSpec(memory_space=pltpu.MemorySpace.VMEM),
        ],
        out_specs=(
            pl.BlockSpec(memory_space=pltpu.MemorySpace.VMEM),
            pl.BlockSpec(memory_space=pltpu.MemorySpace.VMEM),
        ),
        out_shape=(
            jax.ShapeDtypeStruct((_TB, _E), jnp.float32),
            jax.ShapeDtypeStruct((_NSC, _E), jnp.float32),
        ),
        scratch_shapes=[
            pltpu.VMEM((_NBUF, _CB, _S, _E), jnp.float32),
            pltpu.VMEM((_E + _T, _E), jnp.float32),
            pltpu.VMEM((_TB, _E), jnp.float32),
            pltpu.SemaphoreType.DMA((_NBUF,)),
            pltpu.SemaphoreType.DMA,
        ],
    )(x, text_embedding, W1, b1r)

    weights, indices, logits = pl.pallas_call(
        _epilogue_kernel,
        in_specs=[
            pl.BlockSpec(memory_space=pltpu.MemorySpace.VMEM),
            pl.BlockSpec(memory_space=pltpu.MemorySpace.VMEM),
            pl.BlockSpec(memory_space=pltpu.MemorySpace.VMEM),
            pl.BlockSpec(memory_space=pl.ANY),
            pl.BlockSpec(memory_space=pltpu.MemorySpace.VMEM),
            pl.BlockSpec(memory_space=pltpu.MemorySpace.VMEM),
        ],
        out_specs=(
            pl.BlockSpec(memory_space=pltpu.MemorySpace.VMEM),
            pl.BlockSpec(memory_space=pltpu.MemorySpace.VMEM),
            pl.BlockSpec(memory_space=pltpu.MemorySpace.VMEM),
        ),
        out_shape=(
            jax.ShapeDtypeStruct((_B, _K), jnp.float32),
            jax.ShapeDtypeStruct((_B, _K), jnp.int32),
            jax.ShapeDtypeStruct((_B, _NE), jnp.float32),
        ),
        scratch_shapes=[
            pltpu.VMEM((_E, _E), jnp.float32),
            pltpu.SemaphoreType.DMA,
        ],
    )(h_tc, ht_sc, sc_partials, W1, W2, b2r)
    return (weights, indices, logits)


# speedup vs baseline: 1.3668x; 1.2365x over previous
import jax
import jax.numpy as jnp
from jax.experimental import pallas as pl
from jax.experimental.pallas import tpu as pltpu

_B, _S, _E = 64, 1024, 1024
_T = 768
_NE = 16
_K = 2
_NBUF = 4
_CB = 2
_NCH = _B // _CB


def _gate_kernel(x_hbm, text_ref, w1_hbm, b1_ref, w2_ref, b2_ref,
                 w_out_ref, i_out_ref, l_out_ref,
                 buf, w1_v, acc_ref, ht_ref, h_ref, sems, w1_sem):
    for r in range(_NBUF):
        pltpu.make_async_copy(
            x_hbm.at[pl.ds(r * _CB, _CB)], buf.at[r], sems.at[r]).start()
    pltpu.make_async_copy(w1_hbm, w1_v, w1_sem).start()

    def outer(o, carry):
        @pl.when(o == 2)
        def _text_stage():
            pltpu.make_async_copy(w1_hbm, w1_v, w1_sem).wait()
            w1b = w1_v[_E:_E + _T, :]
            ht_ref[...] = (jnp.dot(text_ref[...], w1b,
                                   preferred_element_type=jnp.float32)
                           + b1_ref[...])

        @pl.when(o == 6)
        def _partial_h_stage():
            w1a = w1_v[0:_E, :]
            mean0 = acc_ref[0:48, :] * (1.0 / _S)
            h0 = (jnp.dot(mean0, w1a, preferred_element_type=jnp.float32)
                  + ht_ref[0:48, :])
            h_ref[0:48, :] = jnp.maximum(h0, 0.0)

        for r in range(_NBUF):
            c = o * _NBUF + r
            pltpu.make_async_copy(
                x_hbm.at[pl.ds(c * _CB, _CB)], buf.at[r], sems.at[r]).wait()
            s = jnp.sum(buf[r], axis=1)
            for q in range(_CB):
                acc_ref[pl.ds(c * _CB + q, 1), :] = s[q:q + 1]
            nc = c + _NBUF

            @pl.when(nc < _NCH)
            def _():
                pltpu.make_async_copy(
                    x_hbm.at[pl.ds(nc * _CB, _CB)], buf.at[r], sems.at[r]).start()
        return carry

    jax.lax.fori_loop(0, _NCH // _NBUF, outer, 0)

    w1a = w1_v[0:_E, :]
    mean1 = acc_ref[48:_B, :] * (1.0 / _S)
    h1 = (jnp.dot(mean1, w1a, preferred_element_type=jnp.float32)
          + ht_ref[48:_B, :])
    h_ref[48:_B, :] = jnp.maximum(h1, 0.0)
    logits = (jnp.dot(h_ref[...], w2_ref[...],
                      preferred_element_type=jnp.float32)
              + b2_ref[...])
    l_out_ref[...] = logits

    lane = jax.lax.broadcasted_iota(jnp.int32, (_B, _NE), 1)
    m1 = jnp.max(logits, axis=1, keepdims=True)
    i1 = jnp.min(jnp.where(logits == m1, lane, _NE), axis=1, keepdims=True)
    masked = jnp.where(lane == i1, -jnp.inf, logits)
    m2 = jnp.max(masked, axis=1, keepdims=True)
    i2 = jnp.min(jnp.where(masked == m2, lane, _NE), axis=1, keepdims=True)

    lane2 = jax.lax.broadcasted_iota(jnp.int32, (_B, _K), 1)
    i_out_ref[...] = jnp.where(lane2 == 0, i1, i2)
    e2 = jnp.exp(m2 - m1)
    denom = 1.0 + e2
    w_out_ref[...] = jnp.where(lane2 == 0, 1.0 / denom, e2 / denom)


def kernel(x, text_embedding, W1, b1, W2, b2):
    b1r = b1.reshape(1, _E)
    b2r = b2.reshape(1, _NE)
    out_shape = (
        jax.ShapeDtypeStruct((_B, _K), jnp.float32),
        jax.ShapeDtypeStruct((_B, _K), jnp.int32),
        jax.ShapeDtypeStruct((_B, _NE), jnp.float32),
    )
    weights, indices, logits = pl.pallas_call(
        _gate_kernel,
        in_specs=[
            pl.BlockSpec(memory_space=pl.ANY),
            pl.BlockSpec(memory_space=pltpu.MemorySpace.VMEM),
            pl.BlockSpec(memory_space=pl.ANY),
            pl.BlockSpec(memory_space=pltpu.MemorySpace.VMEM),
            pl.BlockSpec(memory_space=pltpu.MemorySpace.VMEM),
            pl.BlockSpec(memory_space=pltpu.MemorySpace.VMEM),
        ],
        out_specs=(
            pl.BlockSpec(memory_space=pltpu.MemorySpace.VMEM),
            pl.BlockSpec(memory_space=pltpu.MemorySpace.VMEM),
            pl.BlockSpec(memory_space=pltpu.MemorySpace.VMEM),
        ),
        out_shape=out_shape,
        scratch_shapes=[
            pltpu.VMEM((_NBUF, _CB, _S, _E), jnp.float32),
            pltpu.VMEM((_E + _T, _E), jnp.float32),
            pltpu.VMEM((_B, _E), jnp.float32),
            pltpu.VMEM((_B, _E), jnp.float32),
            pltpu.VMEM((_B, _E), jnp.float32),
            pltpu.SemaphoreType.DMA((_NBUF,)),
            pltpu.SemaphoreType.DMA,
        ],
    )(x, text_embedding, W1, b1r, W2, b2r)
    return (weights, indices, logits)
